# baseline (speedup 1.0000x reference)
import functools

import jax
import jax.numpy as jnp
from jax import lax
from jax.experimental import pallas as pl
from jax.experimental.pallas import tpu as pltpu
from jax.experimental.pallas import tpu_sc as plsc

N_NODES = 10000
D_FEAT = 128
N_EDGES = 320000
N_CLASSES = 40

NC = 2
NS = 16
NW = NC * NS

GCH = 128
NCH = 80
SPAN = NCH // 2
EPW = NCH * GCH
PAD_EDGES = NW * EPW - N_EDGES
ROWS_PER_SUBCORE = 624
TAIL_ROWS = N_NODES - NS * ROWS_PER_SUBCORE
TAIL_BASE = NS * ROWS_PER_SUBCORE


def _sc_partial_agg(xz, src, dst, zeros):
  mesh = plsc.VectorSubcoreMesh(core_axis_name="c", subcore_axis_name="s")

  @functools.partial(
      pl.kernel,
      out_type=jax.ShapeDtypeStruct((NC, N_NODES, D_FEAT), jnp.float32),
      mesh=mesh,
      scratch_types=[
          pltpu.VMEM((SPAN, GCH), jnp.int32),
          pltpu.VMEM((NCH, GCH), jnp.int32),
          pltpu.VMEM((GCH, D_FEAT), jnp.float32),
          pltpu.VMEM((GCH, D_FEAT), jnp.float32),
          pltpu.VMEM_SHARED((N_NODES, D_FEAT), jnp.float32),
          pltpu.SemaphoreType.DMA,
          pltpu.SemaphoreType.DMA,
      ],
  )
  def k(x_hbm, src_hbm, dst_hbm, zeros_hbm, out_hbm, sidx, didx, rows_a,
        rows_b, accum, sem_a, sem_b):
    cid = lax.axis_index("c")
    sid = lax.axis_index("s")
    wid = sid * NC + cid
    rbase = sid * ROWS_PER_SUBCORE

    pltpu.sync_copy(src_hbm.at[wid, pl.ds(0, SPAN)], sidx)
    pltpu.sync_copy(dst_hbm.at[wid], didx)
    pltpu.sync_copy(zeros_hbm.at[pl.ds(rbase, ROWS_PER_SUBCORE)],
                    accum.at[pl.ds(rbase, ROWS_PER_SUBCORE)])

    @pl.when(sid == NS - 1)
    def _():
      pltpu.sync_copy(zeros_hbm.at[pl.ds(TAIL_BASE, TAIL_ROWS)],
                      accum.at[pl.ds(TAIL_BASE, TAIL_ROWS)])

    plsc.subcore_barrier()

    def gather(c, base, rows, sem):
      pltpu.async_copy(x_hbm.at[sidx.at[c - base]], rows, sem)

    def drain(c, base, rows, sem):
      pltpu.make_async_copy(x_hbm.at[sidx.at[c - base]], rows, sem).wait()

    def scatter(c, rows):
      pltpu.sync_copy(rows, accum.at[didx.at[c]], add=True)

    def span(base):
      gather(base, base, rows_a, sem_a)

      def body(j):
        c = base + 2 * j
        gather(c + 1, base, rows_b, sem_b)
        drain(c, base, rows_a, sem_a)
        scatter(c, rows_a)
        gather(c + 2, base, rows_a, sem_a)
        drain(c + 1, base, rows_b, sem_b)
        scatter(c + 1, rows_b)

      pl.loop(0, SPAN // 2 - 1)(body)
      last = base + SPAN - 1
      gather(last, base, rows_b, sem_b)
      drain(last - 1, base, rows_a, sem_a)
      scatter(last - 1, rows_a)
      drain(last, base, rows_b, sem_b)
      scatter(last, rows_b)

    span(0)
    pltpu.sync_copy(src_hbm.at[wid, pl.ds(SPAN, SPAN)], sidx)
    span(SPAN)
    plsc.subcore_barrier()

    pltpu.sync_copy(accum.at[pl.ds(rbase, ROWS_PER_SUBCORE)],
                    out_hbm.at[cid, pl.ds(rbase, ROWS_PER_SUBCORE)])

    @pl.when(sid == NS - 1)
    def _():
      pltpu.sync_copy(accum.at[pl.ds(TAIL_BASE, TAIL_ROWS)],
                      out_hbm.at[cid, pl.ds(TAIL_BASE, TAIL_ROWS)])

  return k(xz, src, dst, zeros)


def _tc_mlp_body(x_ref, a0_ref, a1_ref, eps_ref, w1t_ref, b1_ref, gamma_ref,
                 beta_ref, w2t_ref, b2_ref, out_ref):
  h = (1.0 + eps_ref[0, 0]) * x_ref[...] + a0_ref[...] + a1_ref[...]
  h = jnp.dot(h, w1t_ref[...], preferred_element_type=jnp.float32)
  h = jnp.maximum(h + b1_ref[...], 0.0)
  mean = jnp.mean(h, axis=0, keepdims=True)
  var = jnp.mean(jnp.square(h - mean), axis=0, keepdims=True)
  h = (h - mean) * lax.rsqrt(var + 1e-5) * gamma_ref[...] + beta_ref[...]
  o = jnp.dot(h, w2t_ref[...], preferred_element_type=jnp.float32)
  o = o + b2_ref[...]
  m = jnp.max(o, axis=-1, keepdims=True)
  lse = m + jnp.log(jnp.sum(jnp.exp(o - m), axis=-1, keepdims=True))
  out_ref[...] = o - lse


def _tc_mlp(x, a0, a1, eps, w1t, b1, gamma, beta, w2t, b2):
  return pl.pallas_call(
      _tc_mlp_body,
      out_shape=jax.ShapeDtypeStruct((N_NODES, N_CLASSES), jnp.float32),
      in_specs=[
          pl.BlockSpec(memory_space=pltpu.VMEM),
          pl.BlockSpec(memory_space=pltpu.VMEM),
          pl.BlockSpec(memory_space=pltpu.VMEM),
          pl.BlockSpec(memory_space=pltpu.SMEM),
          pl.BlockSpec(memory_space=pltpu.VMEM),
          pl.BlockSpec(memory_space=pltpu.VMEM),
          pl.BlockSpec(memory_space=pltpu.VMEM),
          pl.BlockSpec(memory_space=pltpu.VMEM),
          pl.BlockSpec(memory_space=pltpu.VMEM),
          pl.BlockSpec(memory_space=pltpu.VMEM),
      ],
      out_specs=pl.BlockSpec(memory_space=pltpu.VMEM),
  )(x, a0, a1, eps, w1t, b1, gamma, beta, w2t, b2)


def kernel(x, edge_index, eps, W1, b1, gamma, beta, W2, b2):
  src = jnp.concatenate([
      edge_index[0].astype(jnp.int32),
      jnp.full((PAD_EDGES,), N_NODES, jnp.int32),
  ]).reshape(NW, NCH, GCH)
  dst = jnp.concatenate([
      edge_index[1].astype(jnp.int32),
      jnp.arange(PAD_EDGES, dtype=jnp.int32) % N_NODES,
  ]).reshape(NW, NCH, GCH)
  xz = jnp.concatenate([x, jnp.zeros((1, D_FEAT), jnp.float32)], axis=0)
  zeros = jnp.zeros((N_NODES, D_FEAT), jnp.float32)
  agg = _sc_partial_agg(xz, src, dst, zeros)
  eps2d = jnp.reshape(eps.astype(jnp.float32), (1, 1))
  out = _tc_mlp(x, agg[0], agg[1], eps2d, W1.T, jnp.reshape(b1, (1, -1)),
                jnp.reshape(gamma, (1, -1)), jnp.reshape(beta, (1, -1)),
                W2.T, jnp.reshape(b2, (1, -1)))
  return out


# speedup vs baseline: 2.5725x; 2.5725x over previous
import functools

import jax
import jax.numpy as jnp
from jax import lax
from jax.experimental import pallas as pl
from jax.experimental.pallas import tpu as pltpu
from jax.experimental.pallas import tpu_sc as plsc

N_NODES = 10000
D_FEAT = 128
N_EDGES = 320000
N_CLASSES = 40

NC = 2
NS = 16
NW = NC * NS

GCH = 128
NCH = 80
SPAN = NCH // 2
EPW = NCH * GCH
REAL_PER_WORKER = N_EDGES // NW
PAD_PER_WORKER = EPW - REAL_PER_WORKER
ZPAD = 128
ROWS_PER_SUBCORE = 624
TAIL_ROWS = N_NODES - NS * ROWS_PER_SUBCORE
TAIL_BASE = NS * ROWS_PER_SUBCORE


def _sc_partial_agg(xz, src, dst, zeros):
  mesh = plsc.VectorSubcoreMesh(core_axis_name="c", subcore_axis_name="s")

  @functools.partial(
      pl.kernel,
      out_type=jax.ShapeDtypeStruct((NC, N_NODES, D_FEAT), jnp.float32),
      mesh=mesh,
      scratch_types=[
          pltpu.VMEM((SPAN, GCH), jnp.int32),
          pltpu.VMEM((NCH, GCH), jnp.int32),
          pltpu.VMEM((GCH, D_FEAT), jnp.float32),
          pltpu.VMEM((GCH, D_FEAT), jnp.float32),
          pltpu.VMEM_SHARED((N_NODES, D_FEAT), jnp.float32),
          pltpu.SemaphoreType.DMA,
          pltpu.SemaphoreType.DMA,
      ],
  )
  def k(x_hbm, src_hbm, dst_hbm, zeros_hbm, out_hbm, sidx, didx, rows_a,
        rows_b, accum, sem_a, sem_b):
    cid = lax.axis_index("c")
    sid = lax.axis_index("s")
    wid = sid * NC + cid
    rbase = sid * ROWS_PER_SUBCORE

    pltpu.sync_copy(src_hbm.at[wid, pl.ds(0, SPAN)], sidx)
    pltpu.sync_copy(dst_hbm.at[wid], didx)
    pltpu.sync_copy(zeros_hbm.at[pl.ds(rbase, ROWS_PER_SUBCORE)],
                    accum.at[pl.ds(rbase, ROWS_PER_SUBCORE)])

    @pl.when(sid == NS - 1)
    def _():
      pltpu.sync_copy(zeros_hbm.at[pl.ds(TAIL_BASE, TAIL_ROWS)],
                      accum.at[pl.ds(TAIL_BASE, TAIL_ROWS)])

    plsc.subcore_barrier()

    def gather(c, base, rows, sem):
      pltpu.async_copy(x_hbm.at[sidx.at[c - base]], rows, sem)

    def drain(c, base, rows, sem):
      pltpu.make_async_copy(x_hbm.at[sidx.at[c - base]], rows, sem).wait()

    def scatter(c, rows):
      pltpu.sync_copy(rows, accum.at[didx.at[c]], add=True)

    def span(base):
      gather(base, base, rows_a, sem_a)

      def body(j):
        c = base + 2 * j
        gather(c + 1, base, rows_b, sem_b)
        drain(c, base, rows_a, sem_a)
        scatter(c, rows_a)
        gather(c + 2, base, rows_a, sem_a)
        drain(c + 1, base, rows_b, sem_b)
        scatter(c + 1, rows_b)

      pl.loop(0, SPAN // 2 - 1)(body)
      last = base + SPAN - 1
      gather(last, base, rows_b, sem_b)
      drain(last - 1, base, rows_a, sem_a)
      scatter(last - 1, rows_a)
      drain(last, base, rows_b, sem_b)
      scatter(last, rows_b)

    span(0)
    pltpu.sync_copy(src_hbm.at[wid, pl.ds(SPAN, SPAN)], sidx)
    span(SPAN)
    plsc.subcore_barrier()

    pltpu.sync_copy(accum.at[pl.ds(rbase, ROWS_PER_SUBCORE)],
                    out_hbm.at[cid, pl.ds(rbase, ROWS_PER_SUBCORE)])

    @pl.when(sid == NS - 1)
    def _():
      pltpu.sync_copy(accum.at[pl.ds(TAIL_BASE, TAIL_ROWS)],
                      out_hbm.at[cid, pl.ds(TAIL_BASE, TAIL_ROWS)])

  return k(xz, src, dst, zeros)


def _tc_mlp_body(x_ref, a0_ref, a1_ref, eps_ref, w1t_ref, b1_ref, gamma_ref,
                 beta_ref, w2t_ref, b2_ref, out_ref):
  h = (1.0 + eps_ref[0, 0]) * x_ref[...] + a0_ref[...] + a1_ref[...]
  h = jnp.dot(h, w1t_ref[...], preferred_element_type=jnp.float32)
  h = jnp.maximum(h + b1_ref[...], 0.0)
  mean = jnp.mean(h, axis=0, keepdims=True)
  var = jnp.mean(jnp.square(h - mean), axis=0, keepdims=True)
  h = (h - mean) * lax.rsqrt(var + 1e-5) * gamma_ref[...] + beta_ref[...]
  o = jnp.dot(h, w2t_ref[...], preferred_element_type=jnp.float32)
  o = o + b2_ref[...]
  m = jnp.max(o, axis=-1, keepdims=True)
  lse = m + jnp.log(jnp.sum(jnp.exp(o - m), axis=-1, keepdims=True))
  out_ref[...] = o - lse


def _tc_mlp(x, a0, a1, eps, w1t, b1, gamma, beta, w2t, b2):
  return pl.pallas_call(
      _tc_mlp_body,
      out_shape=jax.ShapeDtypeStruct((N_NODES, N_CLASSES), jnp.float32),
      in_specs=[
          pl.BlockSpec(memory_space=pltpu.VMEM),
          pl.BlockSpec(memory_space=pltpu.VMEM),
          pl.BlockSpec(memory_space=pltpu.VMEM),
          pl.BlockSpec(memory_space=pltpu.SMEM),
          pl.BlockSpec(memory_space=pltpu.VMEM),
          pl.BlockSpec(memory_space=pltpu.VMEM),
          pl.BlockSpec(memory_space=pltpu.VMEM),
          pl.BlockSpec(memory_space=pltpu.VMEM),
          pl.BlockSpec(memory_space=pltpu.VMEM),
          pl.BlockSpec(memory_space=pltpu.VMEM),
      ],
      out_specs=pl.BlockSpec(memory_space=pltpu.VMEM),
  )(x, a0, a1, eps, w1t, b1, gamma, beta, w2t, b2)


def kernel(x, edge_index, eps, W1, b1, gamma, beta, W2, b2):
  pad_iota = jnp.arange(NW * PAD_PER_WORKER, dtype=jnp.int32).reshape(
      NW, PAD_PER_WORKER)
  src = jnp.concatenate([
      edge_index[0].astype(jnp.int32).reshape(NW, REAL_PER_WORKER),
      N_NODES + pad_iota % ZPAD,
  ], axis=1).reshape(NW, NCH, GCH)
  dst = jnp.concatenate([
      edge_index[1].astype(jnp.int32).reshape(NW, REAL_PER_WORKER),
      pad_iota % N_NODES,
  ], axis=1).reshape(NW, NCH, GCH)
  xz = jnp.concatenate([x, jnp.zeros((ZPAD, D_FEAT), jnp.float32)], axis=0)
  zeros = jnp.zeros((N_NODES, D_FEAT), jnp.float32)
  agg = _sc_partial_agg(xz, src, dst, zeros)
  eps2d = jnp.reshape(eps.astype(jnp.float32), (1, 1))
  out = _tc_mlp(x, agg[0], agg[1], eps2d, W1.T, jnp.reshape(b1, (1, -1)),
                jnp.reshape(gamma, (1, -1)), jnp.reshape(beta, (1, -1)),
                W2.T, jnp.reshape(b2, (1, -1)))
  return out


# speedup vs baseline: 2.6570x; 1.0328x over previous
import functools

import jax
import jax.numpy as jnp
from jax import lax
from jax.experimental import pallas as pl
from jax.experimental.pallas import tpu as pltpu
from jax.experimental.pallas import tpu_sc as plsc

N_NODES = 10000
D_FEAT = 128
N_EDGES = 320000
N_CLASSES = 40

NC = 2
NS = 16
NW = NC * NS

GCH = 128
NCH = 80
SPAN = NCH // 2
EPW = NCH * GCH
REAL_PER_WORKER = N_EDGES // NW
PAD_PER_WORKER = EPW - REAL_PER_WORKER
ZPAD = 128
ROWS_PER_SUBCORE = 624
TAIL_ROWS = N_NODES - NS * ROWS_PER_SUBCORE
TAIL_BASE = NS * ROWS_PER_SUBCORE


def _sc_partial_agg(xz, src, dst):
  mesh = plsc.VectorSubcoreMesh(core_axis_name="c", subcore_axis_name="s")

  @functools.partial(
      pl.kernel,
      out_type=jax.ShapeDtypeStruct((NC, N_NODES, D_FEAT), jnp.float32),
      mesh=mesh,
      scratch_types=[
          pltpu.VMEM((SPAN, GCH), jnp.int32),
          pltpu.VMEM((NCH, GCH), jnp.int32),
          pltpu.VMEM((GCH, D_FEAT), jnp.float32),
          pltpu.VMEM((GCH, D_FEAT), jnp.float32),
          pltpu.VMEM_SHARED((N_NODES, D_FEAT), jnp.float32),
          pltpu.SemaphoreType.DMA,
          pltpu.SemaphoreType.DMA,
      ],
  )
  def k(x_hbm, src_hbm, dst_hbm, out_hbm, sidx, didx, rows_a,
        rows_b, accum, sem_a, sem_b):
    cid = lax.axis_index("c")
    sid = lax.axis_index("s")
    wid = sid * NC + cid
    rbase = sid * ROWS_PER_SUBCORE

    pltpu.async_copy(src_hbm.at[wid, pl.ds(0, SPAN)], sidx, sem_a)
    pltpu.async_copy(dst_hbm.at[wid], didx, sem_b)

    z16 = jnp.zeros((16,), jnp.float32)

    def zrow(r):
      for c16 in range(D_FEAT // 16):
        rows_a[r, pl.ds(c16 * 16, 16)] = z16

    pl.loop(0, GCH)(zrow)
    for blk in range(ROWS_PER_SUBCORE // GCH):
      pltpu.sync_copy(rows_a, accum.at[pl.ds(rbase + blk * GCH, GCH)])
    rem = ROWS_PER_SUBCORE % GCH
    pltpu.sync_copy(rows_a.at[pl.ds(0, rem)],
                    accum.at[pl.ds(rbase + ROWS_PER_SUBCORE - rem, rem)])

    @pl.when(sid == NS - 1)
    def _():
      pltpu.sync_copy(rows_a.at[pl.ds(0, TAIL_ROWS)],
                      accum.at[pl.ds(TAIL_BASE, TAIL_ROWS)])

    pltpu.make_async_copy(src_hbm.at[wid, pl.ds(0, SPAN)], sidx, sem_a).wait()
    pltpu.make_async_copy(dst_hbm.at[wid], didx, sem_b).wait()
    plsc.subcore_barrier()

    def gather(c, base, rows, sem):
      pltpu.async_copy(x_hbm.at[sidx.at[c - base]], rows, sem)

    def drain(c, base, rows, sem):
      pltpu.make_async_copy(x_hbm.at[sidx.at[c - base]], rows, sem).wait()

    def scatter(c, rows):
      pltpu.sync_copy(rows, accum.at[didx.at[c]], add=True)

    def span(base):
      gather(base, base, rows_a, sem_a)

      def body(j):
        c = base + 2 * j
        gather(c + 1, base, rows_b, sem_b)
        drain(c, base, rows_a, sem_a)
        scatter(c, rows_a)
        gather(c + 2, base, rows_a, sem_a)
        drain(c + 1, base, rows_b, sem_b)
        scatter(c + 1, rows_b)

      pl.loop(0, SPAN // 2 - 1)(body)
      last = base + SPAN - 1
      gather(last, base, rows_b, sem_b)
      drain(last - 1, base, rows_a, sem_a)
      scatter(last - 1, rows_a)
      drain(last, base, rows_b, sem_b)
      scatter(last, rows_b)

    span(0)
    pltpu.sync_copy(src_hbm.at[wid, pl.ds(SPAN, SPAN)], sidx)
    span(SPAN)
    plsc.subcore_barrier()

    pltpu.sync_copy(accum.at[pl.ds(rbase, ROWS_PER_SUBCORE)],
                    out_hbm.at[cid, pl.ds(rbase, ROWS_PER_SUBCORE)])

    @pl.when(sid == NS - 1)
    def _():
      pltpu.sync_copy(accum.at[pl.ds(TAIL_BASE, TAIL_ROWS)],
                      out_hbm.at[cid, pl.ds(TAIL_BASE, TAIL_ROWS)])

  return k(xz, src, dst)


def _tc_xw1_body(x_ref, w1t_ref, out_ref):
  out_ref[...] = jnp.dot(x_ref[...], w1t_ref[...],
                         preferred_element_type=jnp.float32)


def _tc_xw1(x, w1t):
  return pl.pallas_call(
      _tc_xw1_body,
      out_shape=jax.ShapeDtypeStruct((N_NODES, D_FEAT), jnp.float32),
      in_specs=[
          pl.BlockSpec(memory_space=pltpu.VMEM),
          pl.BlockSpec(memory_space=pltpu.VMEM),
      ],
      out_specs=pl.BlockSpec(memory_space=pltpu.VMEM),
  )(x, w1t)


def _tc_mlp_body(xw1_ref, a0_ref, a1_ref, eps_ref, w1t_ref, b1_ref, gamma_ref,
                 beta_ref, w2t_ref, b2_ref, out_ref):
  agg = a0_ref[...] + a1_ref[...]
  h = ((1.0 + eps_ref[0, 0]) * xw1_ref[...]
       + jnp.dot(agg, w1t_ref[...], preferred_element_type=jnp.float32))
  h = jnp.maximum(h + b1_ref[...], 0.0)
  mean = jnp.mean(h, axis=0, keepdims=True)
  var = jnp.mean(jnp.square(h - mean), axis=0, keepdims=True)
  h = (h - mean) * lax.rsqrt(var + 1e-5) * gamma_ref[...] + beta_ref[...]
  o = jnp.dot(h, w2t_ref[...], preferred_element_type=jnp.float32)
  o = o + b2_ref[...]
  m = jnp.max(o, axis=-1, keepdims=True)
  lse = m + jnp.log(jnp.sum(jnp.exp(o - m), axis=-1, keepdims=True))
  out_ref[...] = o - lse


def _tc_mlp(x, a0, a1, eps, w1t, b1, gamma, beta, w2t, b2):
  return pl.pallas_call(
      _tc_mlp_body,
      out_shape=jax.ShapeDtypeStruct((N_NODES, N_CLASSES), jnp.float32),
      in_specs=[
          pl.BlockSpec(memory_space=pltpu.VMEM),
          pl.BlockSpec(memory_space=pltpu.VMEM),
          pl.BlockSpec(memory_space=pltpu.VMEM),
          pl.BlockSpec(memory_space=pltpu.SMEM),
          pl.BlockSpec(memory_space=pltpu.VMEM),
          pl.BlockSpec(memory_space=pltpu.VMEM),
          pl.BlockSpec(memory_space=pltpu.VMEM),
          pl.BlockSpec(memory_space=pltpu.VMEM),
          pl.BlockSpec(memory_space=pltpu.VMEM),
          pl.BlockSpec(memory_space=pltpu.VMEM),
      ],
      out_specs=pl.BlockSpec(memory_space=pltpu.VMEM),
  )(x, a0, a1, eps, w1t, b1, gamma, beta, w2t, b2)


def kernel(x, edge_index, eps, W1, b1, gamma, beta, W2, b2):
  pad_iota = jnp.arange(NW * PAD_PER_WORKER, dtype=jnp.int32).reshape(
      NW, PAD_PER_WORKER)
  src = jnp.concatenate([
      edge_index[0].astype(jnp.int32).reshape(NW, REAL_PER_WORKER),
      N_NODES + pad_iota % ZPAD,
  ], axis=1).reshape(NW, NCH, GCH)
  dst = jnp.concatenate([
      edge_index[1].astype(jnp.int32).reshape(NW, REAL_PER_WORKER),
      pad_iota % N_NODES,
  ], axis=1).reshape(NW, NCH, GCH)
  xz = jnp.concatenate([x, jnp.zeros((ZPAD, D_FEAT), jnp.float32)], axis=0)
  w1t = W1.T
  xw1 = _tc_xw1(x, w1t)
  agg = _sc_partial_agg(xz, src, dst)
  eps2d = jnp.reshape(eps.astype(jnp.float32), (1, 1))
  out = _tc_mlp(xw1, agg[0], agg[1], eps2d, w1t, jnp.reshape(b1, (1, -1)),
                jnp.reshape(gamma, (1, -1)), jnp.reshape(beta, (1, -1)),
                W2.T, jnp.reshape(b2, (1, -1)))
  return out


# speedup vs baseline: 2.6800x; 1.0087x over previous
import functools

import jax
import jax.numpy as jnp
from jax import lax
from jax.experimental import pallas as pl
from jax.experimental.pallas import tpu as pltpu
from jax.experimental.pallas import tpu_sc as plsc

N_NODES = 10000
D_FEAT = 128
N_EDGES = 320000
N_CLASSES = 40

NC = 2
NS = 16
NW = NC * NS

GCH = 128
NCH = 80
SPAN = NCH // 2
EPW = NCH * GCH
REAL_PER_WORKER = N_EDGES // NW
PAD_PER_WORKER = EPW - REAL_PER_WORKER
ZPAD = 128
ROWS_PER_SUBCORE = 624
TAIL_ROWS = N_NODES - NS * ROWS_PER_SUBCORE
TAIL_BASE = NS * ROWS_PER_SUBCORE


def _sc_partial_agg(xz, src, dst):
  mesh = plsc.VectorSubcoreMesh(core_axis_name="c", subcore_axis_name="s")

  @functools.partial(
      pl.kernel,
      out_type=jax.ShapeDtypeStruct((NC, N_NODES, D_FEAT), jnp.float32),
      mesh=mesh,
      scratch_types=[
          pltpu.VMEM((SPAN, GCH), jnp.int32),
          pltpu.VMEM((NCH, GCH), jnp.int32),
          pltpu.VMEM((GCH, D_FEAT), jnp.float32),
          pltpu.VMEM((GCH, D_FEAT), jnp.float32),
          pltpu.VMEM_SHARED((N_NODES, D_FEAT), jnp.float32),
          pltpu.SemaphoreType.DMA,
          pltpu.SemaphoreType.DMA,
      ],
  )
  def k(x_hbm, src_hbm, dst_hbm, out_hbm, sidx, didx, rows_a,
        rows_b, accum, sem_a, sem_b):
    cid = lax.axis_index("c")
    sid = lax.axis_index("s")
    wid = sid * NC + cid
    rbase = sid * ROWS_PER_SUBCORE

    pltpu.async_copy(src_hbm.at[wid, pl.ds(0, SPAN)], sidx, sem_a)
    pltpu.async_copy(dst_hbm.at[wid], didx, sem_b)

    z16 = jnp.zeros((16,), jnp.float32)

    def zrow(r):
      for c16 in range(D_FEAT // 16):
        rows_a[r, pl.ds(c16 * 16, 16)] = z16

    pl.loop(0, GCH)(zrow)
    nblk = ROWS_PER_SUBCORE // GCH
    rem = ROWS_PER_SUBCORE % GCH
    for blk in range(nblk):
      pltpu.async_copy(rows_a, accum.at[pl.ds(rbase + blk * GCH, GCH)],
                       sem_b)
    pltpu.async_copy(rows_a.at[pl.ds(0, rem)],
                     accum.at[pl.ds(rbase + ROWS_PER_SUBCORE - rem, rem)],
                     sem_b)

    @pl.when(sid == NS - 1)
    def _():
      pltpu.sync_copy(rows_a.at[pl.ds(0, TAIL_ROWS)],
                      accum.at[pl.ds(TAIL_BASE, TAIL_ROWS)])

    for blk in range(nblk):
      pltpu.make_async_copy(rows_a, accum.at[pl.ds(rbase + blk * GCH, GCH)],
                            sem_b).wait()
    pltpu.make_async_copy(rows_a.at[pl.ds(0, rem)],
                          accum.at[pl.ds(rbase + ROWS_PER_SUBCORE - rem, rem)],
                          sem_b).wait()
    pltpu.make_async_copy(src_hbm.at[wid, pl.ds(0, SPAN)], sidx, sem_a).wait()
    pltpu.make_async_copy(dst_hbm.at[wid], didx, sem_b).wait()
    plsc.subcore_barrier()

    def gather(c, base, rows, sem):
      pltpu.async_copy(x_hbm.at[sidx.at[c - base]], rows, sem)

    def drain(c, base, rows, sem):
      pltpu.make_async_copy(x_hbm.at[sidx.at[c - base]], rows, sem).wait()

    def scatter(c, rows):
      pltpu.sync_copy(rows, accum.at[didx.at[c]], add=True)

    def span(base):
      gather(base, base, rows_a, sem_a)

      def body(j):
        c = base + 2 * j
        gather(c + 1, base, rows_b, sem_b)
        drain(c, base, rows_a, sem_a)
        scatter(c, rows_a)
        gather(c + 2, base, rows_a, sem_a)
        drain(c + 1, base, rows_b, sem_b)
        scatter(c + 1, rows_b)

      pl.loop(0, SPAN // 2 - 1)(body)
      last = base + SPAN - 1
      gather(last, base, rows_b, sem_b)
      drain(last - 1, base, rows_a, sem_a)
      scatter(last - 1, rows_a)
      drain(last, base, rows_b, sem_b)
      scatter(last, rows_b)

    span(0)
    pltpu.sync_copy(src_hbm.at[wid, pl.ds(SPAN, SPAN)], sidx)
    span(SPAN)
    plsc.subcore_barrier()

    pltpu.sync_copy(accum.at[pl.ds(rbase, ROWS_PER_SUBCORE)],
                    out_hbm.at[cid, pl.ds(rbase, ROWS_PER_SUBCORE)])

    @pl.when(sid == NS - 1)
    def _():
      pltpu.sync_copy(accum.at[pl.ds(TAIL_BASE, TAIL_ROWS)],
                      out_hbm.at[cid, pl.ds(TAIL_BASE, TAIL_ROWS)])

  return k(xz, src, dst)


def _tc_mlp_body(x_ref, a0_ref, a1_ref, eps_ref, w1t_ref, b1_ref, gamma_ref,
                 beta_ref, w2t_ref, b2_ref, out_ref):
  h = (1.0 + eps_ref[0, 0]) * x_ref[...] + a0_ref[...] + a1_ref[...]
  h = jnp.dot(h, w1t_ref[...], preferred_element_type=jnp.float32)
  h = jnp.maximum(h + b1_ref[...], 0.0)
  mean = jnp.mean(h, axis=0, keepdims=True)
  var = jnp.mean(jnp.square(h - mean), axis=0, keepdims=True)
  h = (h - mean) * lax.rsqrt(var + 1e-5) * gamma_ref[...] + beta_ref[...]
  o = jnp.dot(h, w2t_ref[...], preferred_element_type=jnp.float32)
  o = o + b2_ref[...]
  m = jnp.max(o, axis=-1, keepdims=True)
  lse = m + jnp.log(jnp.sum(jnp.exp(o - m), axis=-1, keepdims=True))
  out_ref[...] = o - lse


def _tc_mlp(x, a0, a1, eps, w1t, b1, gamma, beta, w2t, b2):
  return pl.pallas_call(
      _tc_mlp_body,
      out_shape=jax.ShapeDtypeStruct((N_NODES, N_CLASSES), jnp.float32),
      in_specs=[
          pl.BlockSpec(memory_space=pltpu.VMEM),
          pl.BlockSpec(memory_space=pltpu.VMEM),
          pl.BlockSpec(memory_space=pltpu.VMEM),
          pl.BlockSpec(memory_space=pltpu.SMEM),
          pl.BlockSpec(memory_space=pltpu.VMEM),
          pl.BlockSpec(memory_space=pltpu.VMEM),
          pl.BlockSpec(memory_space=pltpu.VMEM),
          pl.BlockSpec(memory_space=pltpu.VMEM),
          pl.BlockSpec(memory_space=pltpu.VMEM),
          pl.BlockSpec(memory_space=pltpu.VMEM),
      ],
      out_specs=pl.BlockSpec(memory_space=pltpu.VMEM),
  )(x, a0, a1, eps, w1t, b1, gamma, beta, w2t, b2)


def kernel(x, edge_index, eps, W1, b1, gamma, beta, W2, b2):
  pad_iota = jnp.arange(NW * PAD_PER_WORKER, dtype=jnp.int32).reshape(
      NW, PAD_PER_WORKER)
  src = jnp.concatenate([
      edge_index[0].astype(jnp.int32).reshape(NW, REAL_PER_WORKER),
      N_NODES + pad_iota % ZPAD,
  ], axis=1).reshape(NW, NCH, GCH)
  dst = jnp.concatenate([
      edge_index[1].astype(jnp.int32).reshape(NW, REAL_PER_WORKER),
      pad_iota % N_NODES,
  ], axis=1).reshape(NW, NCH, GCH)
  xz = jnp.concatenate([x, jnp.zeros((ZPAD, D_FEAT), jnp.float32)], axis=0)
  agg = _sc_partial_agg(xz, src, dst)
  eps2d = jnp.reshape(eps.astype(jnp.float32), (1, 1))
  out = _tc_mlp(x, agg[0], agg[1], eps2d, W1.T, jnp.reshape(b1, (1, -1)),
                jnp.reshape(gamma, (1, -1)), jnp.reshape(beta, (1, -1)),
                W2.T, jnp.reshape(b2, (1, -1)))
  return out
